# baseline (speedup 1.0000x reference)
import functools

import jax
import jax.numpy as jnp
from jax.experimental import pallas as pl


def _gate_kernel(x_ref, wt_ref, b_ref, probs_ref, ent_ref):
    x = x_ref[...]
    logits = jnp.dot(x, wt_ref[...], preferred_element_type=jnp.float32)
    logits = logits + b_ref[...]
    m = jnp.max(logits, axis=-1, keepdims=True)
    e = jnp.exp(logits - m)
    s = jnp.sum(e, axis=-1, keepdims=True)
    p = e / s
    probs_ref[...] = p
    plogp = p * jnp.log(p + 1e-9)
    ent_ref[...] = jnp.sum(plogp, axis=0, keepdims=True)[None]


@functools.partial(jax.jit, static_argnames=("block",))
def _switch_gate(x, W, b, block=512):
    tokens, in_dim = x.shape
    num_experts = W.shape[0]
    wt = W.T
    b2 = b.reshape(1, num_experts)
    nb = tokens // block
    probs, ent_parts = pl.pallas_call(
        _gate_kernel,
        grid=(nb,),
        in_specs=[
            pl.BlockSpec((block, in_dim), lambda i: (i, 0)),
            pl.BlockSpec((in_dim, num_experts), lambda i: (0, 0)),
            pl.BlockSpec((1, num_experts), lambda i: (0, 0)),
        ],
        out_specs=[
            pl.BlockSpec((block, num_experts), lambda i: (i, 0)),
            pl.BlockSpec((1, 1, num_experts), lambda i: (i, 0, 0)),
        ],
        out_shape=[
            jax.ShapeDtypeStruct((tokens, num_experts), jnp.float32),
            jax.ShapeDtypeStruct((nb, 1, num_experts), jnp.float32),
        ],
    )(x, wt, b2)
    gate_entropy = -(jnp.sum(ent_parts) / tokens)
    return probs, gate_entropy


def kernel(x, W, b):
    return _switch_gate(x, W, b)
